# Initial kernel scaffold; baseline (speedup 1.0000x reference)
import jax
import jax.numpy as jnp
from jax.experimental import pallas as pl


def kernel(dist, path_data, edge_emb, attn_map_weights):
    raise NotImplementedError("write your pallas kernel here")


# speedup vs baseline: 9.1218x; 9.1218x over previous
import functools

import jax
import jax.numpy as jnp
from jax import lax
from jax.experimental import pallas as pl
from jax.experimental.pallas import tpu as pltpu
from jax.experimental.pallas import tpu_sc as plsc

B, N, L, D = 8, 64, 4, 4
NUM_EDGES, FEAT, HEADS = 100000, 64, 8
V = NUM_EDGES + 1
GRP = 32
ROWS = B * N * N
SLOTS = L * D

ROW_BLK = 2048
NBLK = (V + ROW_BLK - 1) // ROW_BLK
VP = NBLK * ROW_BLK

NW = 32
ROWS_PER_W = ROWS // NW
CHUNK = 128
NCHUNK = ROWS_PER_W // CHUNK


def _table_body(e_ref, w_ref, o_ref):
    o_ref[...] = (
        jnp.dot(e_ref[...], w_ref[...], preferred_element_type=jnp.float32) * 0.25
    )


def _build_table(edge_emb, w128):
    return pl.pallas_call(
        _table_body,
        grid=(NBLK,),
        in_specs=[
            pl.BlockSpec((ROW_BLK, FEAT), lambda i: (i, 0)),
            pl.BlockSpec((FEAT, L * GRP), lambda i: (0, 0)),
        ],
        out_specs=pl.BlockSpec((ROW_BLK, L * GRP), lambda i: (i, 0)),
        out_shape=jax.ShapeDtypeStruct((VP, L * GRP), jnp.float32),
    )(edge_emb, w128)


def _gather_body(table_hbm, idx_hbm, out_hbm, idx_v, acc_v, out_v, sem_g):
    wid = lax.axis_index("s") * 2 + lax.axis_index("c")
    pltpu.sync_copy(idx_hbm.at[wid], idx_v)

    for c in range(NCHUNK):
        first = [
            pltpu.async_copy(
                table_hbm.at[idx_v.at[c, 4 * l]], acc_v.at[l], sem_g
            )
            for l in range(L)
        ]
        for cp in first:
            cp.wait()
        rest = [
            pltpu.async_copy(
                table_hbm.at[idx_v.at[c, 4 * l + d]], acc_v.at[l], sem_g,
                add=True,
            )
            for l in range(L)
            for d in range(1, D)
        ]
        for cp in rest:
            cp.wait()

        def row_step(r, carry):
            out_v[r, :] = (
                (acc_v[0, r, pl.ds(0 * GRP, 16)] + acc_v[1, r, pl.ds(1 * GRP, 16)])
                + (acc_v[2, r, pl.ds(2 * GRP, 16)] + acc_v[3, r, pl.ds(3 * GRP, 16)])
            )
            return carry

        lax.fori_loop(0, CHUNK, row_step, 0)
        pltpu.sync_copy(
            out_v, out_hbm.at[pl.ds(wid * ROWS_PER_W + c * CHUNK, CHUNK)]
        )


@functools.partial(
    pl.kernel,
    out_type=jax.ShapeDtypeStruct((ROWS, 16), jnp.float32),
    mesh=plsc.VectorSubcoreMesh(core_axis_name="c", subcore_axis_name="s"),
    scratch_types=[
        pltpu.VMEM((NCHUNK, SLOTS, CHUNK), jnp.int32),
        pltpu.VMEM((L, CHUNK, L * GRP), jnp.float32),
        pltpu.VMEM((CHUNK, 16), jnp.float32),
        pltpu.SemaphoreType.DMA,
    ],
)
def _gather_sum(table_hbm, idx_hbm, out_hbm, idx_v, acc_v, out_v, sem_g):
    _gather_body(table_hbm, idx_hbm, out_hbm, idx_v, acc_v, out_v, sem_g)


def kernel(dist, path_data, edge_emb, attn_map_weights):
    max_len = path_data.shape[-2]
    w128 = jnp.pad(attn_map_weights[:max_len], ((0, 0), (0, 0), (0, GRP - HEADS)))
    w128 = w128.transpose(1, 0, 2).reshape(FEAT, L * GRP)
    table = _build_table(edge_emb, w128)
    idx = (
        path_data.reshape(NW, NCHUNK, CHUNK, SLOTS)
        .transpose(0, 1, 3, 2)
    )
    out = _gather_sum(table, idx)
    out = out[:, :HEADS].reshape(B, N, N, HEADS) / dist[..., None].astype(jnp.float32)
    return out
